# Initial kernel scaffold; baseline (speedup 1.0000x reference)
import jax
import jax.numpy as jnp
from jax.experimental import pallas as pl


def kernel(x, table):
    raise NotImplementedError("write your pallas kernel here")


# speedup vs baseline: 1.8728x; 1.8728x over previous
import functools

import jax
import jax.numpy as jnp
from jax import lax
from jax.experimental import pallas as pl
from jax.experimental.pallas import tpu as pltpu
from jax.experimental.pallas import tpu_sc as plsc

VOCAB = 1000000
N_EMBD = 64
BATCH = 16384
HIST = 50

NC = 2
NS = 16
NW = NC * NS
B_TOTAL = BATCH * HIST
BPW = B_TOTAL // NW
CHUNK = 128
CPG = 4
GROUP = CHUNK * CPG
NCHUNK = BPW // CHUNK
NGROUP = BPW // GROUP
NBUF = 2

assert B_TOTAL % NW == 0 and BPW % GROUP == 0

_mesh = plsc.VectorSubcoreMesh(core_axis_name="c", subcore_axis_name="s")


@functools.partial(
    pl.kernel,
    out_type=jax.ShapeDtypeStruct((B_TOTAL, N_EMBD), jnp.float32),
    mesh=_mesh,
    compiler_params=pltpu.CompilerParams(use_tc_tiling_on_sc=False),
    scratch_types=[
        pltpu.VMEM((NCHUNK, CHUNK), jnp.int32),
        pltpu.VMEM((NBUF, GROUP, N_EMBD), jnp.float32),
        pltpu.SemaphoreType.DMA,
        pltpu.SemaphoreType.DMA,
    ],
)
def _embed_lookup(idx_hbm, table_hbm, out_hbm, idx_v, rows_v, gsem, osem):
    wid = lax.axis_index("s") * NC + lax.axis_index("c")
    base = wid * BPW
    pltpu.sync_copy(idx_hbm.at[wid], idx_v)

    def fire_group(g, buf):
        for c in range(CPG):
            pltpu.async_copy(
                table_hbm.at[idx_v.at[g * CPG + c]],
                rows_v.at[buf, pl.ds(c * CHUNK, CHUNK)],
                gsem,
            )

    def drain_group(buf):
        for c in range(CPG):
            pltpu.make_async_copy(
                table_hbm.at[idx_v.at[0]],
                rows_v.at[buf, pl.ds(c * CHUNK, CHUNK)],
                gsem,
            ).wait()

    def put_group(g, buf):
        pltpu.async_copy(
            rows_v.at[buf],
            out_hbm.at[pl.ds(base + g * GROUP, GROUP)],
            osem,
        )

    def wait_put(buf):
        pltpu.make_async_copy(
            rows_v.at[buf],
            out_hbm.at[pl.ds(0, GROUP)],
            osem,
        ).wait()

    fire_group(0, 0)
    fire_group(1, 1)

    def body(g2, _):
        for b in range(NBUF):
            g = g2 * NBUF + b
            drain_group(b)
            put_group(g, b)
            @pl.when(g + NBUF < NGROUP)
            def _():
                wait_put(b)
                fire_group(g + NBUF, b)
        return 0

    lax.fori_loop(0, NGROUP // NBUF, body, 0)
    wait_put(0)
    wait_put(1)


def kernel(x, table):
    idx = jnp.reshape(x.astype(jnp.int32), (NW, NCHUNK, CHUNK))
    out = _embed_lookup(idx, table)
    return jnp.reshape(out, (BATCH, HIST, N_EMBD))


# speedup vs baseline: 1.8755x; 1.0014x over previous
import functools

import jax
import jax.numpy as jnp
from jax import lax
from jax.experimental import pallas as pl
from jax.experimental.pallas import tpu as pltpu
from jax.experimental.pallas import tpu_sc as plsc

VOCAB = 1000000
N_EMBD = 64
BATCH = 16384
HIST = 50

NC = 2
NS = 16
NW = NC * NS
B_TOTAL = BATCH * HIST
BPW = B_TOTAL // NW
CHUNK = 128
CPG = 2
GROUP = CHUNK * CPG
NCHUNK = BPW // CHUNK
NGROUP = BPW // GROUP
NBUF = 5
FIRE_AHEAD = 3

assert B_TOTAL % NW == 0 and BPW % GROUP == 0 and NGROUP % NBUF == 0
assert FIRE_AHEAD <= NBUF - 2

_mesh = plsc.VectorSubcoreMesh(core_axis_name="c", subcore_axis_name="s")


@functools.partial(
    pl.kernel,
    out_type=jax.ShapeDtypeStruct((B_TOTAL, N_EMBD), jnp.float32),
    mesh=_mesh,
    compiler_params=pltpu.CompilerParams(use_tc_tiling_on_sc=False),
    scratch_types=[
        pltpu.VMEM((NCHUNK, CHUNK), jnp.int32),
        pltpu.VMEM((NBUF, GROUP, N_EMBD), jnp.float32),
        pltpu.SemaphoreType.DMA,
        pltpu.SemaphoreType.DMA,
    ],
)
def _embed_lookup(idx_hbm, table_hbm, out_hbm, idx_v, rows_v, gsem, osem):
    wid = lax.axis_index("s") * NC + lax.axis_index("c")
    base = wid * BPW
    pltpu.sync_copy(idx_hbm.at[wid], idx_v)

    def fire_group(g, buf):
        for c in range(CPG):
            pltpu.async_copy(
                table_hbm.at[idx_v.at[g * CPG + c]],
                rows_v.at[buf, pl.ds(c * CHUNK, CHUNK)],
                gsem,
            )

    def drain_group(buf):
        for c in range(CPG):
            pltpu.make_async_copy(
                table_hbm.at[idx_v.at[0]],
                rows_v.at[buf, pl.ds(c * CHUNK, CHUNK)],
                gsem,
            ).wait()

    def put_group(g, buf):
        pltpu.async_copy(
            rows_v.at[buf],
            out_hbm.at[pl.ds(base + g * GROUP, GROUP)],
            osem,
        )

    def wait_put(buf):
        pltpu.make_async_copy(
            rows_v.at[buf],
            out_hbm.at[pl.ds(0, GROUP)],
            osem,
        ).wait()

    for g in range(FIRE_AHEAD):
        fire_group(g, g)

    def body(t, _):
        for b in range(NBUF):
            g = t * NBUF + b
            gf_buf = (b + FIRE_AHEAD) % NBUF

            @pl.when(g + FIRE_AHEAD < NGROUP)
            def _():
                @pl.when(g >= NBUF - FIRE_AHEAD)
                def _():
                    wait_put(gf_buf)

                fire_group(g + FIRE_AHEAD, gf_buf)

            drain_group(b)
            put_group(g, b)
        return 0

    lax.fori_loop(0, NGROUP // NBUF, body, 0)
    for b in range(NBUF):
        wait_put(b)


def kernel(x, table):
    idx = jnp.reshape(x.astype(jnp.int32), (NW, NCHUNK, CHUNK))
    out = _embed_lookup(idx, table)
    return jnp.reshape(out, (BATCH, HIST, N_EMBD))
